# Initial kernel scaffold; baseline (speedup 1.0000x reference)
import jax
import jax.numpy as jnp
from jax.experimental import pallas as pl


def kernel(X, ref_a, backref, e_map, v_count, W, W_prop, b):
    raise NotImplementedError("write your pallas kernel here")


# speedup vs baseline: 6.6637x; 6.6637x over previous
import functools

import jax
import jax.numpy as jnp
from jax import lax
from jax.experimental import pallas as pl
from jax.experimental.pallas import tpu as pltpu
from jax.experimental.pallas import tpu_sc as plsc

N_NODES = 10000
N_EDGES = 320000
D = 128

NC = 2
NS = 16
EC = N_EDGES // NC
ES = EC // NS
WIN = 128
NFULL = ES // WIN
TAIL = ES - NFULL * WIN
NPS_A = 632
NPS_B = N_NODES - (NS - 1) * NPS_A


def _sc_gather_segment_sum(x, ref_a, backref):
    mesh = plsc.VectorSubcoreMesh(core_axis_name="c", subcore_axis_name="s")

    @functools.partial(
        pl.kernel,
        out_type=jax.ShapeDtypeStruct((NC, N_NODES, D), jnp.float32),
        mesh=mesh,
        scratch_types=[
            pltpu.VMEM_SHARED((N_NODES, D), jnp.float32),
            pltpu.VMEM((WIN,), jnp.int32),
            pltpu.VMEM((WIN,), jnp.int32),
            pltpu.VMEM((WIN, D), jnp.float32),
            pltpu.VMEM((TAIL,), jnp.int32),
            pltpu.VMEM((TAIL,), jnp.int32),
            pltpu.VMEM((TAIL, D), jnp.float32),
            pltpu.SemaphoreType.DMA,
        ],
    )
    def k(x_hbm, ra_hbm, br_hbm, out_hbm, acc, idx_v, bidx_v, rows_v,
          idx_t, bidx_t, rows_t, sem):
        c = lax.axis_index("c")
        s = lax.axis_index("s")

        @pl.loop(0, WIN)
        def _(i):
            @pl.loop(0, D, step=16)
            def _(j):
                rows_v[i, pl.ds(j, 16)] = jnp.zeros((16,), jnp.float32)

        nbase = pl.multiple_of(s * NPS_A, 8)

        def zero_rows(base, nrows):
            @pl.loop(0, nrows // WIN)
            def _(t):
                pltpu.sync_copy(rows_v, acc.at[pl.ds(base + t * WIN, WIN)])
            rem = nrows - (nrows // WIN) * WIN
            if rem:
                pltpu.sync_copy(rows_v.at[pl.ds(0, rem)],
                                acc.at[pl.ds(base + (nrows // WIN) * WIN, rem)])

        @pl.when(s < NS - 1)
        def _():
            zero_rows(nbase, NPS_A)

        @pl.when(s == NS - 1)
        def _():
            zero_rows(nbase, NPS_B)

        plsc.subcore_barrier()

        ebase = c * EC + s * ES

        @pl.loop(0, NFULL)
        def _(t):
            off = ebase + t * WIN
            pltpu.sync_copy(ra_hbm.at[pl.ds(off, WIN)], idx_v)
            pltpu.sync_copy(br_hbm.at[pl.ds(off, WIN)], bidx_v)
            pltpu.async_copy(x_hbm.at[idx_v], rows_v, sem).wait()
            pltpu.sync_copy(rows_v, acc.at[bidx_v], add=True)

        off = ebase + NFULL * WIN
        pltpu.sync_copy(ra_hbm.at[pl.ds(off, TAIL)], idx_t)
        pltpu.sync_copy(br_hbm.at[pl.ds(off, TAIL)], bidx_t)
        pltpu.async_copy(x_hbm.at[idx_t], rows_t, sem).wait()
        pltpu.sync_copy(rows_t, acc.at[bidx_t], add=True)

        plsc.subcore_barrier()

        @pl.when(s < NS - 1)
        def _():
            pltpu.sync_copy(acc.at[pl.ds(nbase, NPS_A)],
                            out_hbm.at[c, pl.ds(nbase, NPS_A)])

        @pl.when(s == NS - 1)
        def _():
            pltpu.sync_copy(acc.at[pl.ds(nbase, NPS_B)],
                            out_hbm.at[c, pl.ds(nbase, NPS_B)])

    return k(x, ref_a, backref)


def _tc_combine(x, s0, s1, w, w_prop, b):
    br = 1000

    def body(x_ref, s0_ref, s1_ref, w_ref, wp_ref, b_ref, o_ref):
        acc = jnp.dot(x_ref[...], w_ref[...], preferred_element_type=jnp.float32)
        conv = s0_ref[...] + s1_ref[...]
        acc += jnp.dot(conv, wp_ref[...], preferred_element_type=jnp.float32)
        o_ref[...] = jnp.maximum(acc + b_ref[...], 0.0)

    return pl.pallas_call(
        body,
        grid=(N_NODES // br,),
        in_specs=[
            pl.BlockSpec((br, D), lambda i: (i, 0)),
            pl.BlockSpec((br, D), lambda i: (i, 0)),
            pl.BlockSpec((br, D), lambda i: (i, 0)),
            pl.BlockSpec((D, D), lambda i: (0, 0)),
            pl.BlockSpec((D, D), lambda i: (0, 0)),
            pl.BlockSpec((1, D), lambda i: (0, 0)),
        ],
        out_specs=pl.BlockSpec((br, D), lambda i: (i, 0)),
        out_shape=jax.ShapeDtypeStruct((N_NODES, D), jnp.float32),
    )(x, s0, s1, w, w_prop, b.reshape(1, D))


def kernel(X, ref_a, backref, e_map, v_count, W, W_prop, b):
    partials = _sc_gather_segment_sum(X, ref_a, backref)
    X_out = _tc_combine(X, partials[0], partials[1], W, W_prop, b)
    return (X_out, ref_a, backref, e_map, v_count)
